# baseline (speedup 1.0000x reference)
import functools

import jax
import jax.numpy as jnp
from jax import lax
from jax.experimental import pallas as pl
from jax.experimental.pallas import tpu as pltpu
from jax.experimental.pallas import tpu_sc as plsc

NUM_COLORS = 100000
EMBED_DIM = 128
BATCH = 4096
HIST = 200

_INFO = plsc.get_sparse_core_info()
_NW = _INFO.num_cores * _INFO.num_subcores

_EMBED_W = EMBED_DIM // 2
_B = BATCH * HIST
_B_PER_W = _B // _NW
_CHUNK = 128
_PAIR = 2 * _CHUNK
_N_PAIRS = _B_PER_W // _PAIR
_NBUF = 2
_N_CHUNKS = _B_PER_W // _CHUNK


def _make_kernel():
  mesh = plsc.VectorSubcoreMesh(core_axis_name="c", subcore_axis_name="s")

  @functools.partial(
      pl.kernel,
      out_type=jax.ShapeDtypeStruct((_B, _EMBED_W), jnp.int32),
      mesh=mesh,
      compiler_params=pltpu.CompilerParams(use_tc_tiling_on_sc=False),
      scratch_types=[
          pltpu.VMEM((_N_CHUNKS, _CHUNK), jnp.int32),
          pltpu.VMEM((_NBUF, _PAIR, _EMBED_W), jnp.int32),
      ]
      + [pltpu.SemaphoreType.DMA] * (3 * _NBUF),
  )
  def gather_kernel(table_hbm, idx_hbm, out_hbm, idx_v, rows_v, *sems):
    gsem = sems[: 2 * _NBUF]
    osem = sems[2 * _NBUF:]
    wid = lax.axis_index("s") * _INFO.num_cores + lax.axis_index("c")
    base = wid * _B_PER_W

    pltpu.sync_copy(idx_hbm.at[wid], idx_v)

    def fire_gather(p, b, h):
      pltpu.async_copy(table_hbm.at[idx_v.at[2 * p + h]],
                       rows_v.at[b, pl.ds(h * _CHUNK, _CHUNK)],
                       gsem[2 * b + h])

    def wait_gather(p, b, h):
      pltpu.make_async_copy(table_hbm.at[idx_v.at[2 * p + h]],
                            rows_v.at[b, pl.ds(h * _CHUNK, _CHUNK)],
                            gsem[2 * b + h]).wait()

    def fire_write(p, b):
      pltpu.async_copy(rows_v.at[b],
                       out_hbm.at[pl.ds(base + p * _PAIR, _PAIR)], osem[b])

    def wait_write(p, b):
      pltpu.make_async_copy(rows_v.at[b],
                            out_hbm.at[pl.ds(base + p * _PAIR, _PAIR)],
                            osem[b]).wait()

    @pl.loop(0, _N_PAIRS // _NBUF)
    def block_loop(k):
      p0 = k * _NBUF
      for b in range(_NBUF):
        p = p0 + b

        @pl.when(k > 0)
        def _():
          wait_write(p - _NBUF, b)
        fire_gather(p, b, 0)

        pw = p - 1
        bw = (b - 1) % _NBUF

        @pl.when(pw >= 0)
        def _():
          wait_gather(pw, bw, 0)
          wait_gather(pw, bw, 1)
          fire_write(pw, bw)
        fire_gather(p, b, 1)

    plast = _N_PAIRS - 1
    blast = plast % _NBUF
    wait_gather(plast, blast, 0)
    wait_gather(plast, blast, 1)
    fire_write(plast, blast)
    for b in range(_NBUF):
      wait_write(_N_PAIRS - _NBUF + b, b)

  return gather_kernel


_GATHER = _make_kernel()


@jax.jit
def kernel(color_indices, embedding_table):
  idx = color_indices.astype(jnp.int32).reshape(_NW, _N_CHUNKS, _CHUNK)
  tbl = jax.lax.bitcast_convert_type(
      embedding_table.astype(jnp.bfloat16).reshape(NUM_COLORS, _EMBED_W, 2),
      jnp.int32)
  out = _GATHER(tbl, idx)
  out = jax.lax.bitcast_convert_type(out, jnp.bfloat16)
  return out.reshape(BATCH, HIST, EMBED_DIM).astype(jnp.float32)


# speedup vs baseline: 11.0453x; 11.0453x over previous
import functools

import jax
import jax.numpy as jnp
from jax import lax
from jax.experimental import pallas as pl
from jax.experimental.pallas import tpu as pltpu
from jax.experimental.pallas import tpu_sc as plsc

NUM_COLORS = 100000
EMBED_DIM = 128
BATCH = 4096
HIST = 200

_INFO = plsc.get_sparse_core_info()
_NW = _INFO.num_cores * _INFO.num_subcores

_B = BATCH * HIST
_B_PER_W = _B // _NW
_CHUNK = 128
_N_CHUNKS = _B_PER_W // _CHUNK
_NBUF = 4
_WLAG = 3
_N_BLOCKS = _N_CHUNKS // _NBUF


def _make_kernel():
  mesh = plsc.VectorSubcoreMesh(core_axis_name="c", subcore_axis_name="s")

  @functools.partial(
      pl.kernel,
      out_type=jax.ShapeDtypeStruct((_B, EMBED_DIM), jnp.float32),
      mesh=mesh,
      scratch_types=[
          pltpu.VMEM((_N_CHUNKS, _CHUNK), jnp.int32),
          pltpu.VMEM((_NBUF, _CHUNK, EMBED_DIM), jnp.float32),
      ]
      + [pltpu.SemaphoreType.DMA] * (2 * _NBUF),
  )
  def gather_kernel(table_hbm, idx_hbm, out_hbm, idx_v, rows_v, *sems):
    gsem = sems[:_NBUF]
    osem = sems[_NBUF:]
    wid = lax.axis_index("s") * _INFO.num_cores + lax.axis_index("c")
    base = wid * _B_PER_W

    pltpu.sync_copy(idx_hbm.at[wid], idx_v)

    def fire_gather(j, b):
      pltpu.async_copy(table_hbm.at[idx_v.at[j]], rows_v.at[b], gsem[b])

    def wait_gather(j, b):
      pltpu.make_async_copy(table_hbm.at[idx_v.at[j]], rows_v.at[b],
                            gsem[b]).wait()

    def fire_write(j, b):
      pltpu.async_copy(rows_v.at[b],
                       out_hbm.at[pl.ds(base + j * _CHUNK, _CHUNK)], osem[b])

    def wait_write(j, b):
      pltpu.make_async_copy(rows_v.at[b],
                            out_hbm.at[pl.ds(base + j * _CHUNK, _CHUNK)],
                            osem[b]).wait()

    @pl.loop(0, _N_BLOCKS)
    def block_loop(k):
      j0 = k * _NBUF
      for b in range(_NBUF):
        j = j0 + b

        @pl.when(k > 0)
        def _():
          wait_write(j - _NBUF, b)
        fire_gather(j, b)

        jw = j - _WLAG
        bw = (b - _WLAG) % _NBUF

        @pl.when(jw >= 0)
        def _():
          wait_gather(jw, bw)
          fire_write(jw, bw)

    for d in range(_WLAG):
      jw = _N_CHUNKS - _WLAG + d
      bw = jw % _NBUF
      wait_gather(jw, bw)
      fire_write(jw, bw)
    for b in range(_NBUF):
      wait_write(_N_CHUNKS - _NBUF + b, b)

  return gather_kernel


_GATHER = _make_kernel()


@jax.jit
def kernel(color_indices, embedding_table):
  idx = color_indices.astype(jnp.int32).reshape(_NW, _N_CHUNKS, _CHUNK)
  out = _GATHER(embedding_table, idx)
  return out.reshape(BATCH, HIST, EMBED_DIM)
